# baseline (speedup 1.0000x reference)
import functools

import jax
import jax.numpy as jnp
from jax import lax
from jax.experimental import pallas as pl
from jax.experimental.pallas import tpu as pltpu

EMB = 100
RNN = 100
H1 = 100
NCLS = 2
B = 1024
T = 50
GATE = 128
ZW = 4 * GATE


def _sc_gather(idx_flat, table):
    from jax.experimental.pallas import tpu_sc as plsc

    info = plsc.get_sparse_core_info()
    ncores, nsub = info.num_cores, info.num_subcores
    nw = ncores * nsub
    n = idx_flat.shape[0]
    rows_per_w = n // nw
    chunk = 800
    nchunks = rows_per_w // chunk
    mesh = plsc.VectorSubcoreMesh(core_axis_name="c", subcore_axis_name="s")

    @functools.partial(
        pl.kernel,
        mesh=mesh,
        out_type=jax.ShapeDtypeStruct((n, EMB), jnp.float32),
        scratch_types=[
            pltpu.VMEM((chunk,), jnp.int32),
            pltpu.VMEM((chunk, EMB), jnp.float32),
            pltpu.SemaphoreType.DMA,
        ],
        compiler_params=pltpu.CompilerParams(use_tc_tiling_on_sc=False),
    )
    def gk(idx_hbm, table_hbm, out_hbm, idx_v, rows_v, sem):
        wid = lax.axis_index("s") * ncores + lax.axis_index("c")
        for j in range(nchunks):
            base = wid * rows_per_w + j * chunk
            pltpu.sync_copy(idx_hbm.at[pl.ds(base, chunk)], idx_v)
            pltpu.async_copy(table_hbm.at[idx_v], rows_v, sem).wait()
            pltpu.sync_copy(rows_v, out_hbm.at[pl.ds(base, chunk)])

    return gk(idx_flat, table)


def _lstm_body(x_ref, w_ref, u_ref, b_ref, w1_ref, b1_ref, w15_ref, b15_ref,
               w2_ref, b2_ref, out_ref, h_scr, c_scr):
    t = pl.program_id(0)

    @pl.when(t == 0)
    def _():
        h_scr[...] = jnp.zeros_like(h_scr)
        c_scr[...] = jnp.zeros_like(c_scr)

    x = x_ref[0]
    h = h_scr[...]
    z = jnp.dot(x, w_ref[...], preferred_element_type=jnp.float32)
    z = z + jnp.dot(h, u_ref[...], preferred_element_type=jnp.float32)
    z = z + b_ref[...]
    i = jax.nn.sigmoid(z[:, 0:GATE])
    f = jax.nn.sigmoid(z[:, GATE:2 * GATE])
    g = jnp.tanh(z[:, 2 * GATE:3 * GATE])
    o = jax.nn.sigmoid(z[:, 3 * GATE:4 * GATE])
    c = f * c_scr[...] + i * g
    h_new = o * jnp.tanh(c)
    c_scr[...] = c
    h_scr[...] = h_new

    @pl.when(t == T - 1)
    def _():
        l1 = jnp.maximum(
            jnp.dot(h_new, w1_ref[...], preferred_element_type=jnp.float32)
            + b1_ref[...], 0.0)
        l15 = jnp.maximum(
            jnp.dot(l1, w15_ref[...], preferred_element_type=jnp.float32)
            + b15_ref[...], 0.0)
        logits = (jnp.dot(l15, w2_ref[...], preferred_element_type=jnp.float32)
                  + b2_ref[...])
        m = jnp.max(logits, axis=-1, keepdims=True)
        e = jnp.exp(logits - m)
        out_ref[...] = e / jnp.sum(e, axis=-1, keepdims=True)


def _tc_lstm_mlp(xs, w_pad, u_pad, b_pad, w1_pad, b1, w15, b15, w2, b2):
    return pl.pallas_call(
        _lstm_body,
        grid=(T,),
        in_specs=[
            pl.BlockSpec((1, B, EMB), lambda t: (t, 0, 0)),
            pl.BlockSpec((EMB, ZW), lambda t: (0, 0)),
            pl.BlockSpec((GATE, ZW), lambda t: (0, 0)),
            pl.BlockSpec((1, ZW), lambda t: (0, 0)),
            pl.BlockSpec((GATE, H1), lambda t: (0, 0)),
            pl.BlockSpec((1, H1), lambda t: (0, 0)),
            pl.BlockSpec((H1, H1), lambda t: (0, 0)),
            pl.BlockSpec((1, H1), lambda t: (0, 0)),
            pl.BlockSpec((H1, NCLS), lambda t: (0, 0)),
            pl.BlockSpec((1, NCLS), lambda t: (0, 0)),
        ],
        out_specs=pl.BlockSpec((B, NCLS), lambda t: (0, 0)),
        out_shape=jax.ShapeDtypeStruct((B, NCLS), jnp.float32),
        scratch_shapes=[
            pltpu.VMEM((B, GATE), jnp.float32),
            pltpu.VMEM((B, GATE), jnp.float32),
        ],
        compiler_params=pltpu.CompilerParams(
            dimension_semantics=("arbitrary",)),
    )(xs, w_pad, u_pad, b_pad, w1_pad, b1, w15, b15, w2, b2)


def _pad_gates(w):
    k = w.shape[0]
    w4 = w.reshape(k, 4, RNN)
    w4 = jnp.pad(w4, ((0, 0), (0, 0), (0, GATE - RNN)))
    return w4.reshape(k, 4 * GATE)


def kernel(inputs, E, W_lstm, U_lstm, b_lstm, W1, b1, W15, b15, W2, b2):
    idx = jnp.transpose(inputs).reshape(-1).astype(jnp.int32)
    emb_flat = _sc_gather(idx, E)
    xs = emb_flat.reshape(T, B, EMB)

    w_pad = _pad_gates(W_lstm)
    u_pad = jnp.pad(_pad_gates(U_lstm), ((0, GATE - RNN), (0, 0)))
    b_pad = _pad_gates(b_lstm.reshape(1, -1))
    w1_pad = jnp.pad(W1, ((0, GATE - RNN), (0, 0)))

    return _tc_lstm_mlp(xs, w_pad, u_pad, b_pad, w1_pad,
                        b1.reshape(1, -1), W15, b15.reshape(1, -1),
                        W2, b2.reshape(1, -1))


# speedup vs baseline: 4.7620x; 4.7620x over previous
import functools

import jax
import jax.numpy as jnp
from jax import lax
from jax.experimental import pallas as pl
from jax.experimental.pallas import tpu as pltpu

EMB = 100
RNN = 100
H1 = 100
NCLS = 2
B = 1024
T = 50
GATE = 128
ZW = 4 * GATE


def _sc_gather(idx_flat, table):
    from jax.experimental.pallas import tpu_sc as plsc

    info = plsc.get_sparse_core_info()
    ncores, nsub = info.num_cores, info.num_subcores
    nw = ncores * nsub
    n = idx_flat.shape[0]
    rows_per_w = n // nw
    chunk = 800
    nchunks = rows_per_w // chunk
    groups = chunk // 16
    mesh = plsc.VectorSubcoreMesh(core_axis_name="c", subcore_axis_name="s")

    @functools.partial(
        pl.kernel,
        mesh=mesh,
        out_type=jax.ShapeDtypeStruct((n, EMB), jnp.float32),
        scratch_types=[
            pltpu.VMEM((chunk,), jnp.int32),
            pltpu.VMEM((chunk, EMB), jnp.float32),
            pltpu.SemaphoreType.DMA,
            pltpu.SemaphoreType.DMA,
        ],
    )
    def gk(idx_hbm, table_hbm, out_hbm, idx_v, rows_v, gsem, osem):
        wid = lax.axis_index("s") * ncores + lax.axis_index("c")
        for j in range(nchunks):
            base = wid * rows_per_w + j * chunk
            if j > 0:
                pltpu.make_async_copy(
                    rows_v, out_hbm.at[pl.ds(0, chunk)], osem).wait()
            pltpu.sync_copy(idx_hbm.at[pl.ds(base, chunk)], idx_v)

            def fire_group(g, carry):
                vec = idx_v[pl.ds(g * 16, 16)]
                cps = []
                for l in range(16):
                    r = vec[l]
                    cps.append(pltpu.async_copy(
                        table_hbm.at[pl.ds(r, 1)],
                        rows_v.at[pl.ds(g * 16 + l, 1)],
                        gsem,
                    ))
                for cp in cps:
                    cp.wait()
                return carry

            lax.fori_loop(0, groups, fire_group, 0)
            pltpu.async_copy(rows_v, out_hbm.at[pl.ds(base, chunk)], osem)
        pltpu.make_async_copy(
            rows_v, out_hbm.at[pl.ds(0, chunk)], osem).wait()

    return gk(idx_flat, table)


def _lstm_body(x_ref, w_ref, u_ref, b_ref, w1_ref, b1_ref, w15_ref, b15_ref,
               w2_ref, b2_ref, out_ref, h_scr, c_scr):
    t = pl.program_id(0)

    @pl.when(t == 0)
    def _():
        h_scr[...] = jnp.zeros_like(h_scr)
        c_scr[...] = jnp.zeros_like(c_scr)

    x = x_ref[0]
    h = h_scr[...]
    z = jnp.dot(x, w_ref[...], preferred_element_type=jnp.float32)
    z = z + jnp.dot(h, u_ref[...], preferred_element_type=jnp.float32)
    z = z + b_ref[...]
    i = jax.nn.sigmoid(z[:, 0:GATE])
    f = jax.nn.sigmoid(z[:, GATE:2 * GATE])
    g = jnp.tanh(z[:, 2 * GATE:3 * GATE])
    o = jax.nn.sigmoid(z[:, 3 * GATE:4 * GATE])
    c = f * c_scr[...] + i * g
    h_new = o * jnp.tanh(c)
    c_scr[...] = c
    h_scr[...] = h_new

    @pl.when(t == T - 1)
    def _():
        l1 = jnp.maximum(
            jnp.dot(h_new, w1_ref[...], preferred_element_type=jnp.float32)
            + b1_ref[...], 0.0)
        l15 = jnp.maximum(
            jnp.dot(l1, w15_ref[...], preferred_element_type=jnp.float32)
            + b15_ref[...], 0.0)
        logits = (jnp.dot(l15, w2_ref[...], preferred_element_type=jnp.float32)
                  + b2_ref[...])
        m = jnp.max(logits, axis=-1, keepdims=True)
        e = jnp.exp(logits - m)
        out_ref[...] = e / jnp.sum(e, axis=-1, keepdims=True)


def _tc_lstm_mlp(xs, w_pad, u_pad, b_pad, w1_pad, b1, w15, b15, w2, b2):
    return pl.pallas_call(
        _lstm_body,
        grid=(T,),
        in_specs=[
            pl.BlockSpec((1, B, EMB), lambda t: (t, 0, 0)),
            pl.BlockSpec((EMB, ZW), lambda t: (0, 0)),
            pl.BlockSpec((GATE, ZW), lambda t: (0, 0)),
            pl.BlockSpec((1, ZW), lambda t: (0, 0)),
            pl.BlockSpec((GATE, H1), lambda t: (0, 0)),
            pl.BlockSpec((1, H1), lambda t: (0, 0)),
            pl.BlockSpec((H1, H1), lambda t: (0, 0)),
            pl.BlockSpec((1, H1), lambda t: (0, 0)),
            pl.BlockSpec((H1, NCLS), lambda t: (0, 0)),
            pl.BlockSpec((1, NCLS), lambda t: (0, 0)),
        ],
        out_specs=pl.BlockSpec((B, NCLS), lambda t: (0, 0)),
        out_shape=jax.ShapeDtypeStruct((B, NCLS), jnp.float32),
        scratch_shapes=[
            pltpu.VMEM((B, GATE), jnp.float32),
            pltpu.VMEM((B, GATE), jnp.float32),
        ],
        compiler_params=pltpu.CompilerParams(
            dimension_semantics=("arbitrary",)),
    )(xs, w_pad, u_pad, b_pad, w1_pad, b1, w15, b15, w2, b2)


def _pad_gates(w):
    k = w.shape[0]
    w4 = w.reshape(k, 4, RNN)
    w4 = jnp.pad(w4, ((0, 0), (0, 0), (0, GATE - RNN)))
    return w4.reshape(k, 4 * GATE)


def kernel(inputs, E, W_lstm, U_lstm, b_lstm, W1, b1, W15, b15, W2, b2):
    idx = jnp.transpose(inputs).reshape(-1).astype(jnp.int32)
    emb_flat = _sc_gather(idx, E)
    xs = emb_flat.reshape(T, B, EMB)

    w_pad = _pad_gates(W_lstm)
    u_pad = jnp.pad(_pad_gates(U_lstm), ((0, GATE - RNN), (0, 0)))
    b_pad = _pad_gates(b_lstm.reshape(1, -1))
    w1_pad = jnp.pad(W1, ((0, GATE - RNN), (0, 0)))

    return _tc_lstm_mlp(xs, w_pad, u_pad, b_pad, w1_pad,
                        b1.reshape(1, -1), W15, b15.reshape(1, -1),
                        W2, b2.reshape(1, -1))
